# baseline (speedup 1.0000x reference)
import functools

import jax
import jax.numpy as jnp
from jax import lax
from jax.experimental import pallas as pl
from jax.experimental.pallas import tpu as pltpu
from jax.experimental.pallas import tpu_sc as plsc

B = 4096
L = 200
D = 64
SCALE = 8.0

_HALF = L // 2


def kernel(x, table, pos_encoding):
    info = plsc.get_sparse_core_info()
    nc, ns = info.num_cores, info.num_subcores
    nw = nc * ns
    seq_per_w = B // nw

    mesh = plsc.VectorSubcoreMesh(core_axis_name="c", subcore_axis_name="s")

    @functools.partial(
        pl.kernel,
        mesh=mesh,
        out_type=jax.ShapeDtypeStruct((B, L, D), jnp.float32),
        scratch_types=[
            pltpu.VMEM((2, _HALF), jnp.int32),
            pltpu.VMEM((L, D), jnp.float32),
            pltpu.VMEM((L, D), jnp.float32),
            pltpu.SemaphoreType.DMA,
        ],
        compiler_params=pltpu.CompilerParams(use_tc_tiling_on_sc=False),
    )
    def sc_kernel(x_hbm, pos_hbm, table_hbm, out_hbm, idx_v, pos_v, emb_v, sem):
        wid = lax.axis_index("s") * nc + lax.axis_index("c")
        s0 = wid * seq_per_w

        pltpu.sync_copy(pos_hbm, pos_v)

        def seq_body(i, carry):
            s = s0 + i
            pltpu.sync_copy(x_hbm.at[s], idx_v)
            pltpu.async_copy(
                table_hbm.at[idx_v.at[0]], emb_v.at[pl.ds(0, _HALF)], sem
            ).wait()
            pltpu.async_copy(
                table_hbm.at[idx_v.at[1]], emb_v.at[pl.ds(_HALF, _HALF)], sem
            ).wait()

            def row_body(r, c):
                for j in range(D // 16):
                    sl = pl.ds(j * 16, 16)
                    emb_v[r, sl] = emb_v[r, sl] * SCALE + pos_v[r, sl]
                return c

            lax.fori_loop(0, L, row_body, 0)
            pltpu.sync_copy(emb_v, out_hbm.at[s])
            return carry

        lax.fori_loop(0, seq_per_w, seq_body, 0)

    x3 = x.astype(jnp.int32).reshape(B, 2, _HALF)
    return sc_kernel(x3, pos_encoding, table)


# speedup vs baseline: 1.2571x; 1.2571x over previous
import functools

import jax
import jax.numpy as jnp
from jax import lax
from jax.experimental import pallas as pl
from jax.experimental.pallas import tpu as pltpu
from jax.experimental.pallas import tpu_sc as plsc

B = 4096
L = 200
D = 64
SCALE = 8.0

_HALF = L // 2
_NBUF = 4


def kernel(x, table, pos_encoding):
    info = plsc.get_sparse_core_info()
    nc, ns = info.num_cores, info.num_subcores
    nw = nc * ns
    seq_per_w = B // nw

    mesh = plsc.VectorSubcoreMesh(core_axis_name="c", subcore_axis_name="s")

    @functools.partial(
        pl.kernel,
        mesh=mesh,
        out_type=jax.ShapeDtypeStruct((B, L, D), jnp.float32),
        scratch_types=[
            pltpu.VMEM((seq_per_w, 2, _HALF), jnp.int32),
            pltpu.VMEM((L, D), jnp.float32),
            pltpu.VMEM((_NBUF, L, D), jnp.float32),
            [pltpu.SemaphoreType.DMA] * _NBUF,
            [pltpu.SemaphoreType.DMA] * _NBUF,
        ],
        compiler_params=pltpu.CompilerParams(use_tc_tiling_on_sc=False),
    )
    def sc_kernel(x_hbm, pos_hbm, table_hbm, out_hbm,
                  idx_v, pos_v, emb_v, gsems, osems):
        wid = lax.axis_index("s") * nc + lax.axis_index("c")
        s0 = wid * seq_per_w

        pltpu.sync_copy(x_hbm.at[wid], idx_v)
        pltpu.sync_copy(pos_hbm, pos_v)

        def start_gather(i, b):
            pltpu.async_copy(
                table_hbm.at[idx_v.at[i, 0]], emb_v.at[b, pl.ds(0, _HALF)],
                gsems[b],
            )
            pltpu.async_copy(
                table_hbm.at[idx_v.at[i, 1]], emb_v.at[b, pl.ds(_HALF, _HALF)],
                gsems[b],
            )

        def wait_gather(b):
            pltpu.make_async_copy(
                table_hbm.at[idx_v.at[0, 0]], emb_v.at[b, pl.ds(0, _HALF)],
                gsems[b],
            ).wait()
            pltpu.make_async_copy(
                table_hbm.at[idx_v.at[0, 1]], emb_v.at[b, pl.ds(_HALF, _HALF)],
                gsems[b],
            ).wait()

        def start_store(i, b):
            pltpu.async_copy(emb_v.at[b], out_hbm.at[s0 + i], osems[b])

        def wait_store(b):
            pltpu.make_async_copy(emb_v.at[b], out_hbm.at[s0], osems[b]).wait()

        def compute(b):
            def row_body(r, c):
                for j in range(D // 16):
                    sl = pl.ds(j * 16, 16)
                    emb_v[b, r, sl] = emb_v[b, r, sl] * SCALE + pos_v[r, sl]
                return c

            lax.fori_loop(0, L, row_body, 0)

        start_gather(0, 0)
        start_gather(1, 1)

        def quad_body(k, carry):
            for u in range(_NBUF):
                i = k * _NBUF + u
                wait_gather(u)
                compute(u)

                bn = (u + 2) % _NBUF

                @pl.when(i >= 2)
                def _():
                    wait_store(bn)

                @pl.when(i < seq_per_w - 2)
                def _():
                    start_gather(i + 2, bn)

                start_store(i, u)
            return carry

        lax.fori_loop(0, seq_per_w // _NBUF, quad_body, 0)
        wait_store((seq_per_w - 2) % _NBUF)
        wait_store((seq_per_w - 1) % _NBUF)

    x4 = x.astype(jnp.int32).reshape(nw, seq_per_w, 2, _HALF)
    return sc_kernel(x4, pos_encoding, table)


# speedup vs baseline: 1.2703x; 1.0105x over previous
import functools

import jax
import jax.numpy as jnp
from jax import lax
from jax.experimental import pallas as pl
from jax.experimental.pallas import tpu as pltpu
from jax.experimental.pallas import tpu_sc as plsc

B = 4096
L = 200
D = 64
SCALE = 8.0

_HALF = L // 2
_NBUF = 4


def kernel(x, table, pos_encoding):
    info = plsc.get_sparse_core_info()
    nc, ns = info.num_cores, info.num_subcores
    nw = nc * ns
    seq_per_w = B // nw

    mesh = plsc.VectorSubcoreMesh(core_axis_name="c", subcore_axis_name="s")

    @functools.partial(
        pl.kernel,
        mesh=mesh,
        out_type=jax.ShapeDtypeStruct((B, L, D), jnp.float32),
        scratch_types=[
            pltpu.VMEM((seq_per_w, 2, _HALF), jnp.int32),
            pltpu.VMEM((L, D), jnp.float32),
            pltpu.VMEM((_NBUF, L, D), jnp.float32),
            [pltpu.SemaphoreType.DMA] * _NBUF,
            [pltpu.SemaphoreType.DMA] * _NBUF,
        ],
        compiler_params=pltpu.CompilerParams(use_tc_tiling_on_sc=False),
    )
    def sc_kernel(x_hbm, pos_hbm, table_hbm, out_hbm,
                  idx_v, pos_v, emb_v, gsems, osems):
        wid = lax.axis_index("s") * nc + lax.axis_index("c")
        s0 = wid * seq_per_w

        pltpu.sync_copy(x_hbm.at[wid], idx_v)
        pltpu.sync_copy(pos_hbm, pos_v)

        def start_gather(i, b):
            pltpu.async_copy(
                table_hbm.at[idx_v.at[i, 0]], emb_v.at[b, pl.ds(0, _HALF)],
                gsems[b],
            )
            pltpu.async_copy(
                table_hbm.at[idx_v.at[i, 1]], emb_v.at[b, pl.ds(_HALF, _HALF)],
                gsems[b],
            )

        def wait_gather(b):
            pltpu.make_async_copy(
                table_hbm.at[idx_v.at[0, 0]], emb_v.at[b, pl.ds(0, _HALF)],
                gsems[b],
            ).wait()
            pltpu.make_async_copy(
                table_hbm.at[idx_v.at[0, 1]], emb_v.at[b, pl.ds(_HALF, _HALF)],
                gsems[b],
            ).wait()

        def start_store(i, b):
            pltpu.async_copy(emb_v.at[b], out_hbm.at[s0 + i], osems[b])

        def wait_store(b):
            pltpu.make_async_copy(emb_v.at[b], out_hbm.at[s0], osems[b]).wait()

        def compute(b):
            @plsc.parallel_loop(0, L, step=1, unroll=8)
            def row_body(r):
                for j in range(D // 16):
                    sl = pl.ds(j * 16, 16)
                    emb_v[b, r, sl] = emb_v[b, r, sl] * SCALE + pos_v[r, sl]

        start_gather(0, 0)
        start_gather(1, 1)

        def quad_body(k, carry):
            for u in range(_NBUF):
                i = k * _NBUF + u
                wait_gather(u)
                compute(u)

                bn = (u + 2) % _NBUF

                @pl.when(i >= 2)
                def _():
                    wait_store(bn)

                @pl.when(i < seq_per_w - 2)
                def _():
                    start_gather(i + 2, bn)

                start_store(i, u)
            return carry

        lax.fori_loop(0, seq_per_w // _NBUF, quad_body, 0)
        wait_store((seq_per_w - 2) % _NBUF)
        wait_store((seq_per_w - 1) % _NBUF)

    x4 = x.astype(jnp.int32).reshape(nw, seq_per_w, 2, _HALF)
    return sc_kernel(x4, pos_encoding, table)


# speedup vs baseline: 1.2743x; 1.0032x over previous
import functools

import jax
import jax.numpy as jnp
from jax import lax
from jax.experimental import pallas as pl
from jax.experimental.pallas import tpu as pltpu
from jax.experimental.pallas import tpu_sc as plsc

B = 4096
L = 200
D = 64
SCALE = 8.0

_CHUNKS = ((0, 128), (128, 72))
_NBUF = 4


def kernel(x, table, pos_encoding):
    info = plsc.get_sparse_core_info()
    nc, ns = info.num_cores, info.num_subcores
    nw = nc * ns
    seq_per_w = B // nw

    mesh = plsc.VectorSubcoreMesh(core_axis_name="c", subcore_axis_name="s")

    @functools.partial(
        pl.kernel,
        mesh=mesh,
        out_type=jax.ShapeDtypeStruct((B * L, D), jnp.float32),
        scratch_types=[
            pltpu.VMEM((seq_per_w, L), jnp.int32),
            pltpu.VMEM((L, D), jnp.float32),
            pltpu.VMEM((_NBUF, L, D), jnp.float32),
            [pltpu.SemaphoreType.DMA] * _NBUF,
            [pltpu.SemaphoreType.DMA] * _NBUF,
        ],
        compiler_params=pltpu.CompilerParams(use_tc_tiling_on_sc=False),
    )
    def sc_kernel(x_hbm, pos_hbm, table_hbm, out_hbm,
                  idx_v, pos_v, emb_v, gsems, osems):
        wid = lax.axis_index("s") * nc + lax.axis_index("c")
        s0 = wid * seq_per_w

        pltpu.sync_copy(x_hbm.at[pl.ds(s0, seq_per_w)], idx_v)
        pltpu.sync_copy(pos_hbm, pos_v)

        def start_gather(i, b):
            for off, n in _CHUNKS:
                pltpu.async_copy(
                    table_hbm.at[idx_v.at[i, pl.ds(off, n)]],
                    emb_v.at[b, pl.ds(off, n)],
                    gsems[b],
                )

        def wait_gather(b):
            for off, n in _CHUNKS:
                pltpu.make_async_copy(
                    table_hbm.at[idx_v.at[0, pl.ds(off, n)]],
                    emb_v.at[b, pl.ds(off, n)],
                    gsems[b],
                ).wait()

        def start_store(i, b):
            pltpu.async_copy(
                emb_v.at[b], out_hbm.at[pl.ds((s0 + i) * L, L)], osems[b]
            )

        def wait_store(b):
            pltpu.make_async_copy(
                emb_v.at[b], out_hbm.at[pl.ds(0, L)], osems[b]
            ).wait()

        def compute(b):
            @plsc.parallel_loop(0, L, step=1, unroll=8)
            def row_body(r):
                for j in range(D // 16):
                    sl = pl.ds(j * 16, 16)
                    emb_v[b, r, sl] = emb_v[b, r, sl] * SCALE + pos_v[r, sl]

        start_gather(0, 0)
        start_gather(1, 1)

        def quad_body(k, carry):
            for u in range(_NBUF):
                i = k * _NBUF + u
                wait_gather(u)
                compute(u)

                bn = (u + 2) % _NBUF

                @pl.when(i >= 2)
                def _():
                    wait_store(bn)

                @pl.when(i < seq_per_w - 2)
                def _():
                    start_gather(i + 2, bn)

                start_store(i, u)
            return carry

        lax.fori_loop(0, seq_per_w // _NBUF, quad_body, 0)
        wait_store((seq_per_w - 2) % _NBUF)
        wait_store((seq_per_w - 1) % _NBUF)

    out2d = sc_kernel(x, pos_encoding, table)
    return out2d.reshape(B, L, D)
